# baseline (speedup 1.0000x reference)
import functools

import jax
import jax.numpy as jnp
from jax.experimental import pallas as pl

H = 128
W = 128
HW = H * W
N = 1024
K = 16
THR = 0.01
ABSORP = 1.0
FOCAL = 100.0
PX = 64.0
PY = 64.0

TP = 2048
NEG = -1.0e30
DEAD = -3.0e38
QTHR = 9.210340371976184


def _features():
    yy = (jnp.arange(H, dtype=jnp.float32) - PY) / FOCAL
    xx = (jnp.arange(W, dtype=jnp.float32) - PX) / FOCAL
    y, x = jnp.meshgrid(yy, xx, indexing='ij')
    d = jnp.stack([x, y, jnp.ones_like(x)], axis=-1)
    d = d / jnp.linalg.norm(d, axis=-1, keepdims=True)
    rays = d.reshape(-1, 3)
    rq = (rays[:, :, None] * rays[:, None, :]).reshape(-1, 9)
    zeros = jnp.zeros((HW, 4), dtype=jnp.float32)
    return jnp.concatenate([rq, rays, zeros], axis=1).T


def _tile_kernel(pw_ref, f_ref, w_ref, idx_ref, vnum_ref, len_ref):
    pw = pw_ref[...]
    f = f_ref[...]

    a9 = pw[:, 0:9]
    amu = pw[:, 9:12]
    c = pw[:, 12:13]

    dot = functools.partial(jnp.dot, preferred_element_type=jnp.float32)
    a = dot(a9, f[0:9, :])
    b = dot(amu, f[9:12, :])
    a_safe = jnp.maximum(a, 1e-8)
    t = b / a_safe
    q = jnp.maximum(c - b * b / a_safe, 0.0)
    valid = (q < QTHR) & (t > 0.0)
    key = jnp.where(valid, -q, NEG)

    row = jax.lax.broadcasted_iota(jnp.int32, (N, TP), 0).astype(jnp.float32)

    sel_val, sel_idx, sel_len = [], [], []
    for _ in range(K):
        m = jnp.max(key, axis=0)
        hit = key == m[None, :]
        mi = jnp.min(jnp.where(hit, row, 16384.0), axis=0)
        one = row == mi[None, :]
        tk = jnp.sum(jnp.where(one, t, 0.0), axis=0)
        key = jnp.where(one, DEAD, key)
        sel_val.append(m)
        sel_idx.append(mi)
        sel_len.append(tk)

    vals = jnp.stack(sel_val)
    idxs = jnp.stack(sel_idx)
    lens = jnp.stack(sel_len)

    svalid = vals > (0.5 * NEG)
    svalidf = jnp.where(svalid, 1.0, 0.0)
    acts = jnp.where(svalid, jnp.exp(0.5 * vals), 0.0)

    skey = jnp.where(svalid, lens, 1e9)
    kiota = jax.lax.broadcasted_iota(jnp.int32, (K, TP), 0).astype(jnp.float32)
    s_act, s_idx, s_len, s_vld = [], [], [], []
    for _ in range(K):
        mn = jnp.min(skey, axis=0)
        hit = skey == mn[None, :]
        pos = jnp.min(jnp.where(hit, kiota, float(K)), axis=0)
        one = kiota == pos[None, :]
        onef = jnp.where(one, 1.0, 0.0)
        s_act.append(jnp.sum(onef * acts, axis=0))
        s_idx.append(jnp.sum(onef * idxs, axis=0))
        s_len.append(jnp.sum(onef * lens, axis=0))
        s_vld.append(jnp.sum(onef * svalidf, axis=0))
        skey = jnp.where(one, 2e9, skey)

    trans = jnp.ones((TP,), dtype=jnp.float32)
    ws = []
    for j in range(K):
        alpha = jnp.clip(ABSORP * s_act[j], 0.0, 0.999) * s_vld[j]
        ws.append(alpha * trans)
        trans = trans * (1.0 - alpha)

    w_mat = jnp.stack(ws)
    i_mat = jnp.stack(s_idx)
    l_mat = jnp.stack(s_len)
    vnum = jnp.zeros((TP,), jnp.float32)
    for j in range(K):
        vnum = vnum + s_vld[j]

    w_ref[...] = w_mat.T
    idx_ref[...] = i_mat.T.astype(jnp.int32)
    len_ref[...] = l_mat.T
    vnum_ref[...] = vnum.astype(jnp.int32)[:, None]


def kernel(verts, sigmas, R, T):
    Rm = R[0]
    mu = verts[0] @ Rm + T[0]
    A = 2.0 * (jnp.swapaxes(Rm, 0, 1)[None] @ sigmas[0] @ Rm[None])
    c = jnp.einsum('ni,nij,nj->n', mu, A, mu)
    Amu = jnp.einsum('nij,nj->ni', A, mu)
    A9 = A.reshape(N, 9)
    pw = jnp.concatenate([A9, Amu, c[:, None],
                          jnp.zeros((N, 3), jnp.float32)], axis=1)
    feats = _features()

    grid = HW // TP
    w, idx, vnum, lens = pl.pallas_call(
        _tile_kernel,
        grid=(grid,),
        in_specs=[
            pl.BlockSpec((N, 16), lambda i: (0, 0)),
            pl.BlockSpec((16, TP), lambda i: (0, i)),
        ],
        out_specs=[
            pl.BlockSpec((TP, K), lambda i: (i, 0)),
            pl.BlockSpec((TP, K), lambda i: (i, 0)),
            pl.BlockSpec((TP, 1), lambda i: (i, 0)),
            pl.BlockSpec((TP, K), lambda i: (i, 0)),
        ],
        out_shape=[
            jax.ShapeDtypeStruct((HW, K), jnp.float32),
            jax.ShapeDtypeStruct((HW, K), jnp.int32),
            jax.ShapeDtypeStruct((HW, 1), jnp.int32),
            jax.ShapeDtypeStruct((HW, K), jnp.float32),
        ],
    )(pw, feats)

    return (w.reshape(1, H, W, K), idx.reshape(1, H, W, K),
            vnum.reshape(1, H, W), lens.reshape(1, H, W, K))


# speedup vs baseline: 1.2011x; 1.2011x over previous
import functools

import jax
import jax.numpy as jnp
from jax.experimental import pallas as pl

H = 128
W = 128
HW = H * W
N = 1024
K = 16
THR = 0.01
ABSORP = 1.0
FOCAL = 100.0
PX = 64.0
PY = 64.0

TP = 1024
NEG = -1.0e30
DEAD = -3.0e38
QTHR = 9.210340371976184


def _features():
    yy = (jnp.arange(H, dtype=jnp.float32) - PY) / FOCAL
    xx = (jnp.arange(W, dtype=jnp.float32) - PX) / FOCAL
    y, x = jnp.meshgrid(yy, xx, indexing='ij')
    d = jnp.stack([x, y, jnp.ones_like(x)], axis=-1)
    d = d / jnp.linalg.norm(d, axis=-1, keepdims=True)
    rays = d.reshape(-1, 3)
    rq = (rays[:, :, None] * rays[:, None, :]).reshape(-1, 9)
    zeros = jnp.zeros((HW, 4), dtype=jnp.float32)
    return jnp.concatenate([rq, rays, zeros], axis=1).T


def _tile_kernel(pw_ref, f_ref, w_ref, idx_ref, vnum_ref, len_ref):
    pw = pw_ref[...]
    f = f_ref[...]

    a9 = pw[:, 0:9]
    amu = pw[:, 9:12]
    c = pw[:, 12:13]

    dot = functools.partial(jnp.dot, preferred_element_type=jnp.float32)
    a = dot(a9, f[0:9, :])
    b = dot(amu, f[9:12, :])
    a_safe = jnp.maximum(a, 1e-8)
    t = b / a_safe
    q = jnp.maximum(c - b * b / a_safe, 0.0)
    valid = (q < QTHR) & (t > 0.0)
    key = jnp.where(valid, -q, NEG)

    row = jax.lax.broadcasted_iota(jnp.int32, (N, TP), 0).astype(jnp.float32)

    sel_val, sel_idx, sel_len = [], [], []
    for _ in range(K):
        m = jnp.max(key, axis=0)
        hit = key == m[None, :]
        mi = jnp.min(jnp.where(hit, row, 16384.0), axis=0)
        one = row == mi[None, :]
        tk = jnp.sum(jnp.where(one, t, 0.0), axis=0)
        key = jnp.where(one, DEAD, key)
        sel_val.append(m)
        sel_idx.append(mi)
        sel_len.append(tk)

    vals = jnp.stack(sel_val)
    idxs = jnp.stack(sel_idx)
    lens = jnp.stack(sel_len)

    svalid = vals > (0.5 * NEG)
    svalidf = jnp.where(svalid, 1.0, 0.0)
    acts = jnp.where(svalid, jnp.exp(0.5 * vals), 0.0)

    skey = jnp.where(svalid, lens, 1e9)
    kiota = jax.lax.broadcasted_iota(jnp.int32, (K, TP), 0).astype(jnp.float32)
    s_act, s_idx, s_len, s_vld = [], [], [], []
    for _ in range(K):
        mn = jnp.min(skey, axis=0)
        hit = skey == mn[None, :]
        pos = jnp.min(jnp.where(hit, kiota, float(K)), axis=0)
        one = kiota == pos[None, :]
        onef = jnp.where(one, 1.0, 0.0)
        s_act.append(jnp.sum(onef * acts, axis=0))
        s_idx.append(jnp.sum(onef * idxs, axis=0))
        s_len.append(jnp.sum(onef * lens, axis=0))
        s_vld.append(jnp.sum(onef * svalidf, axis=0))
        skey = jnp.where(one, 2e9, skey)

    trans = jnp.ones((TP,), dtype=jnp.float32)
    ws = []
    for j in range(K):
        alpha = jnp.clip(ABSORP * s_act[j], 0.0, 0.999) * s_vld[j]
        ws.append(alpha * trans)
        trans = trans * (1.0 - alpha)

    w_mat = jnp.stack(ws)
    i_mat = jnp.stack(s_idx)
    l_mat = jnp.stack(s_len)
    vnum = jnp.zeros((TP,), jnp.float32)
    for j in range(K):
        vnum = vnum + s_vld[j]

    w_ref[...] = w_mat.T
    idx_ref[...] = i_mat.T.astype(jnp.int32)
    len_ref[...] = l_mat.T
    vnum_ref[...] = vnum.astype(jnp.int32)[:, None]


def kernel(verts, sigmas, R, T):
    Rm = R[0]
    mu = verts[0] @ Rm + T[0]
    A = 2.0 * (jnp.swapaxes(Rm, 0, 1)[None] @ sigmas[0] @ Rm[None])
    c = jnp.einsum('ni,nij,nj->n', mu, A, mu)
    Amu = jnp.einsum('nij,nj->ni', A, mu)
    A9 = A.reshape(N, 9)
    pw = jnp.concatenate([A9, Amu, c[:, None],
                          jnp.zeros((N, 3), jnp.float32)], axis=1)
    feats = _features()

    grid = HW // TP
    w, idx, vnum, lens = pl.pallas_call(
        _tile_kernel,
        grid=(grid,),
        in_specs=[
            pl.BlockSpec((N, 16), lambda i: (0, 0)),
            pl.BlockSpec((16, TP), lambda i: (0, i)),
        ],
        out_specs=[
            pl.BlockSpec((TP, K), lambda i: (i, 0)),
            pl.BlockSpec((TP, K), lambda i: (i, 0)),
            pl.BlockSpec((TP, 1), lambda i: (i, 0)),
            pl.BlockSpec((TP, K), lambda i: (i, 0)),
        ],
        out_shape=[
            jax.ShapeDtypeStruct((HW, K), jnp.float32),
            jax.ShapeDtypeStruct((HW, K), jnp.int32),
            jax.ShapeDtypeStruct((HW, 1), jnp.int32),
            jax.ShapeDtypeStruct((HW, K), jnp.float32),
        ],
    )(pw, feats)

    return (w.reshape(1, H, W, K), idx.reshape(1, H, W, K),
            vnum.reshape(1, H, W), lens.reshape(1, H, W, K))
